# baseline (speedup 1.0000x reference)
import functools

import jax
import jax.numpy as jnp
import numpy as np
from jax import lax
from jax.experimental import pallas as pl
from jax.experimental.pallas import tpu as pltpu
from jax.experimental.pallas import tpu_sc as plsc

VOCAB = 1000000
EMBED = 32
N_FEATURES = 36
HIDDEN = 200
N_CLASSES = 3
BATCH = 16384

ROWS = BATCH * N_FEATURES
NC, NS = 2, 16
NW = NC * NS
ROWS_PER_W = ROWS // NW
CHUNK = 1024
NCHUNK = ROWS_PER_W // CHUNK
SCAT = CHUNK // 128
PERM_ROWS = ROWS_PER_W // 128
KT = (N_FEATURES * EMBED) // 128


def _perm_table() -> np.ndarray:
    r = np.arange(ROWS, dtype=np.int64)
    b, f = r // N_FEATURES, r % N_FEATURES
    o = ((b // 8) * KT + f // 4) * 32 + (b % 8) * 4 + (f % 4)
    return o.astype(np.int32).reshape(NW, PERM_ROWS, 128)


def _make_sc_gather():
    mesh = plsc.VectorSubcoreMesh(core_axis_name="c", subcore_axis_name="s")

    @functools.partial(
        pl.kernel,
        mesh=mesh,
        out_type=jax.ShapeDtypeStruct((ROWS, EMBED), jnp.bfloat16),
        scratch_types=[
            pltpu.VMEM((ROWS_PER_W,), jnp.int32),
            pltpu.VMEM((PERM_ROWS, 128), jnp.int32),
            pltpu.VMEM((2, CHUNK, EMBED), jnp.bfloat16),
            pltpu.SemaphoreType.DMA,
            pltpu.SemaphoreType.DMA,
        ],
        compiler_params=pltpu.CompilerParams(use_tc_tiling_on_sc=False),
    )
    def gather_k(idx_hbm, table_hbm, perm_hbm, out_hbm,
                 idx_v, perm_v, rows_v, gsem, wsem):
        wid = lax.axis_index("s") * NC + lax.axis_index("c")
        base = wid * ROWS_PER_W
        pltpu.sync_copy(idx_hbm.at[pl.ds(base, ROWS_PER_W)], idx_v)
        pltpu.sync_copy(perm_hbm.at[wid], perm_v)

        def g_start(ci, b):
            pltpu.make_async_copy(
                table_hbm.at[idx_v.at[pl.ds(ci * CHUNK, CHUNK)]],
                rows_v.at[b], gsem,
            ).start()

        def g_wait(b):
            pltpu.make_async_copy(
                table_hbm.at[idx_v.at[pl.ds(0, CHUNK)]], rows_v.at[b], gsem
            ).wait()

        def s_start(ci, b):
            for q in range(SCAT):
                pltpu.make_async_copy(
                    rows_v.at[b].at[pl.ds(q * 128, 128)],
                    out_hbm.at[perm_v.at[ci * SCAT + q]],
                    wsem,
                ).start()

        def s_drain(b):
            for q in range(SCAT):
                pltpu.make_async_copy(
                    rows_v.at[b].at[pl.ds(q * 128, 128)],
                    out_hbm.at[perm_v.at[q]],
                    wsem,
                ).wait()

        g_start(0, 0)
        g_start(1, 1)

        def step(ci, _):
            for b in range(2):
                c = ci + b
                g_wait(b)
                s_start(c, b)
                s_drain(b)
                pl.when(c + 2 < NCHUNK)(lambda: g_start(c + 2, b))
            return 0

        lax.fori_loop(0, NCHUNK // 2, lambda i, c: step(i * 2, c), 0)

    return gather_k


_sc_gather = _make_sc_gather()


TVB = 8192
TG = TVB // 8
NTBLK = -(-VOCAB // TVB)
TROWS = NTBLK * TVB


def _tr_body(et_ref, out_ref):
    t = et_ref[...].astype(jnp.bfloat16).T
    out_ref[...] = jnp.concatenate(
        [t[a * TG:(a + 1) * TG, :] for a in range(8)], axis=1)


def _transpose_table(et):
    return pl.pallas_call(
        _tr_body,
        grid=(NTBLK,),
        in_specs=[pl.BlockSpec((EMBED, TVB), lambda i: (0, i))],
        out_specs=pl.BlockSpec((TG, 256), lambda i: (i, 0)),
        out_shape=jax.ShapeDtypeStruct((NTBLK * TG, 256), jnp.bfloat16),
    )(et).reshape(TROWS, EMBED)


def _remap_idx(idx):
    u = idx & (TVB - 1)
    return 8 * ((idx >> 13) * TG + (u & (TG - 1))) + (u >> 10)


def _mlp_body(x_ref, w1_ref, b1_ref, w2_ref, b2_ref, out_ref):
    bb8 = x_ref.shape[0]
    acc = jnp.zeros((bb8 * 8, HIDDEN), dtype=jnp.float32)
    for t in range(KT):
        xt = x_ref[:, t].reshape(bb8 * 8, 128).astype(jnp.float32)
        acc = acc + jnp.dot(xt, w1_ref[t], preferred_element_type=jnp.float32)
    h = jnp.maximum(acc + b1_ref[...], 0.0)
    out_ref[...] = (
        jnp.dot(h, w2_ref[...], preferred_element_type=jnp.float32) + b2_ref[...]
    )


def _mlp(x4, W1r, b1, W2, b2):
    BB = 1024
    grid = (BATCH // BB,)
    return pl.pallas_call(
        _mlp_body,
        grid=grid,
        in_specs=[
            pl.BlockSpec((BB // 8, KT, 8, 128), lambda i: (i, 0, 0, 0)),
            pl.BlockSpec((KT, 128, HIDDEN), lambda i: (0, 0, 0)),
            pl.BlockSpec((1, HIDDEN), lambda i: (0, 0)),
            pl.BlockSpec((HIDDEN, N_CLASSES), lambda i: (0, 0)),
            pl.BlockSpec((1, N_CLASSES), lambda i: (0, 0)),
        ],
        out_specs=pl.BlockSpec((BB, N_CLASSES), lambda i: (i, 0)),
        out_shape=jax.ShapeDtypeStruct((BATCH, N_CLASSES), jnp.float32),
    )(x4, W1r, b1, W2, b2)


_PERM = _perm_table()


def kernel(w, embeddings, W1, b1, W2, b2):
    idx = _remap_idx(w.reshape(-1))
    table = _transpose_table(embeddings.T)
    xg = _sc_gather(idx, table, jnp.asarray(_PERM))
    x4 = xg.reshape(BATCH // 8, KT, 8, 4, EMBED).reshape(BATCH // 8, KT, 8, 128)
    return _mlp(
        x4,
        W1.reshape(KT, 128, HIDDEN),
        b1.reshape(1, HIDDEN),
        W2,
        b2.reshape(1, N_CLASSES),
    )


# speedup vs baseline: 1.4222x; 1.4222x over previous
import functools

import jax
import jax.numpy as jnp
import numpy as np
from jax import lax
from jax.experimental import pallas as pl
from jax.experimental.pallas import tpu as pltpu
from jax.experimental.pallas import tpu_sc as plsc

VOCAB = 1000000
EMBED = 32
N_FEATURES = 36
HIDDEN = 200
N_CLASSES = 3
BATCH = 16384

ROWS = BATCH * N_FEATURES
NC, NS = 2, 16
NW = NC * NS
ROWS_PER_W = ROWS // NW
CHUNK = 1024
NCHUNK = ROWS_PER_W // CHUNK
SCAT = CHUNK // 128
PERM_ROWS = ROWS_PER_W // 128
KT = (N_FEATURES * EMBED) // 128


def _perm_table() -> np.ndarray:
    r = np.arange(ROWS, dtype=np.int64)
    b, f = r // N_FEATURES, r % N_FEATURES
    o = ((b // 8) * KT + f // 4) * 32 + (b % 8) * 4 + (f % 4)
    return o.astype(np.int32).reshape(NW, PERM_ROWS, 128)


def _make_sc_gather():
    mesh = plsc.VectorSubcoreMesh(core_axis_name="c", subcore_axis_name="s")

    @functools.partial(
        pl.kernel,
        mesh=mesh,
        out_type=jax.ShapeDtypeStruct((ROWS, EMBED), jnp.float32),
        scratch_types=[
            pltpu.VMEM((ROWS_PER_W,), jnp.int32),
            pltpu.VMEM((PERM_ROWS, 128), jnp.int32),
            pltpu.VMEM((2, CHUNK, EMBED), jnp.float32),
            pltpu.SemaphoreType.DMA,
            pltpu.SemaphoreType.DMA,
        ],
        compiler_params=pltpu.CompilerParams(use_tc_tiling_on_sc=False),
    )
    def gather_k(idx_hbm, table_hbm, perm_hbm, out_hbm,
                 idx_v, perm_v, rows_v, gsem, wsem):
        wid = lax.axis_index("s") * NC + lax.axis_index("c")
        base = wid * ROWS_PER_W
        pltpu.sync_copy(idx_hbm.at[pl.ds(base, ROWS_PER_W)], idx_v)
        pltpu.sync_copy(perm_hbm.at[wid], perm_v)

        def g_start(ci, b):
            pltpu.make_async_copy(
                table_hbm.at[idx_v.at[pl.ds(ci * CHUNK, CHUNK)]],
                rows_v.at[b], gsem,
            ).start()

        def g_wait(b):
            pltpu.make_async_copy(
                table_hbm.at[idx_v.at[pl.ds(0, CHUNK)]], rows_v.at[b], gsem
            ).wait()

        def s_start(ci, b):
            for q in range(SCAT):
                pltpu.make_async_copy(
                    rows_v.at[b].at[pl.ds(q * 128, 128)],
                    out_hbm.at[perm_v.at[ci * SCAT + q]],
                    wsem,
                ).start()

        def s_drain(b):
            for q in range(SCAT):
                pltpu.make_async_copy(
                    rows_v.at[b].at[pl.ds(q * 128, 128)],
                    out_hbm.at[perm_v.at[q]],
                    wsem,
                ).wait()

        g_start(0, 0)
        g_start(1, 1)

        def step(ci, _):
            for b in range(2):
                c = ci + b
                g_wait(b)
                s_start(c, b)
                s_drain(b)
                pl.when(c + 2 < NCHUNK)(lambda: g_start(c + 2, b))
            return 0

        lax.fori_loop(0, NCHUNK // 2, lambda i, c: step(i * 2, c), 0)

    return gather_k


_sc_gather = _make_sc_gather()


TVB = 16384
TG = TVB // 4
NTBLK = -(-VOCAB // TVB)
TROWS = NTBLK * TVB


def _tr_body(et_ref, out_ref):
    t = et_ref[...].T
    out_ref[...] = jnp.concatenate(
        [t[a * TG:(a + 1) * TG, :] for a in range(4)], axis=1)


def _transpose_table(et):
    return pl.pallas_call(
        _tr_body,
        grid=(NTBLK,),
        in_specs=[pl.BlockSpec((EMBED, TVB), lambda i: (0, i))],
        out_specs=pl.BlockSpec((TG, 128), lambda i: (i, 0)),
        out_shape=jax.ShapeDtypeStruct((NTBLK * TG, 128), jnp.float32),
    )(et).reshape(TROWS, EMBED)


def _remap_idx(idx):
    u = idx & (TVB - 1)
    return 4 * ((idx >> (TVB.bit_length() - 1)) * TG + (u & (TG - 1))) + (
        u >> (TG.bit_length() - 1))


def _mlp_body(x_ref, w1_ref, b1_ref, w2_ref, b2_ref, out_ref):
    bb8 = x_ref.shape[0]
    acc = jnp.zeros((bb8 * 8, HIDDEN), dtype=jnp.float32)
    for t in range(KT):
        xt = x_ref[:, t].reshape(bb8 * 8, 128)
        acc = acc + jnp.dot(xt, w1_ref[t], preferred_element_type=jnp.float32)
    h = jnp.maximum(acc + b1_ref[...], 0.0)
    out_ref[...] = (
        jnp.dot(h, w2_ref[...], preferred_element_type=jnp.float32) + b2_ref[...]
    )


def _mlp(x4, W1r, b1, W2, b2):
    BB = 1024
    grid = (BATCH // BB,)
    return pl.pallas_call(
        _mlp_body,
        grid=grid,
        in_specs=[
            pl.BlockSpec((BB // 8, KT, 8, 128), lambda i: (i, 0, 0, 0)),
            pl.BlockSpec((KT, 128, HIDDEN), lambda i: (0, 0, 0)),
            pl.BlockSpec((1, HIDDEN), lambda i: (0, 0)),
            pl.BlockSpec((HIDDEN, N_CLASSES), lambda i: (0, 0)),
            pl.BlockSpec((1, N_CLASSES), lambda i: (0, 0)),
        ],
        out_specs=pl.BlockSpec((BB, N_CLASSES), lambda i: (i, 0)),
        out_shape=jax.ShapeDtypeStruct((BATCH, N_CLASSES), jnp.float32),
    )(x4, W1r, b1, W2, b2)


_PERM = _perm_table()


def kernel(w, embeddings, W1, b1, W2, b2):
    idx = _remap_idx(w.reshape(-1))
    table = _transpose_table(embeddings.T)
    xg = _sc_gather(idx, table, jnp.asarray(_PERM))
    x4 = xg.reshape(BATCH // 8, KT, 8, 4, EMBED).reshape(BATCH // 8, KT, 8, 128)
    return _mlp(
        x4,
        W1.reshape(KT, 128, HIDDEN),
        b1.reshape(1, HIDDEN),
        W2,
        b2.reshape(1, N_CLASSES),
    )
